# baseline (speedup 1.0000x reference)
import functools

import jax
import jax.numpy as jnp
from jax.experimental import pallas as pl
from jax.experimental.pallas import tpu as pltpu


NUM_Q = 4
K = 1024
D = 32
BLOCK = 4096


def _rvq_body(xt_ref, cb_ref, cbt_ref, out_ref, loss_ref, *, n_tokens):
    pid = pl.program_id(0)
    r = xt_ref[...]
    qsum = jnp.zeros_like(r)
    acc = jnp.zeros((1, 1), jnp.float32)
    iota_k = jax.lax.broadcasted_iota(jnp.int32, (K, 1), 0)

    for i in range(NUM_Q):
        Wc = cb_ref[i]
        Wt = cbt_ref[i]
        w2 = jnp.sum(Wc * Wc, axis=1, keepdims=True)
        r2 = jnp.sum(r * r, axis=0, keepdims=True)
        r2b = (2.0 * r).astype(jnp.bfloat16)
        scores = jax.lax.dot_general(
            Wc.astype(jnp.bfloat16), r2b, (((1,), (0,)), ((), ())),
            preferred_element_type=jnp.float32)
        dist = (r2 - scores) + w2
        cmin = jnp.min(dist, axis=0, keepdims=True)
        idx = jnp.min(
            jnp.where(dist <= cmin, iota_k, K),
            axis=0, keepdims=True)
        onehot = (iota_k == idx).astype(jnp.bfloat16)
        hi = Wt.astype(jnp.bfloat16)
        rem = Wt - hi.astype(jnp.float32)
        mid = rem.astype(jnp.bfloat16)
        lo = (rem - mid.astype(jnp.float32)).astype(jnp.bfloat16)
        wsplit = jnp.concatenate([hi, mid, lo], axis=0)
        q3 = jax.lax.dot_general(
            wsplit, onehot, (((1,), (0,)), ((), ())),
            preferred_element_type=jnp.float32)
        q = (q3[0:D, :] + q3[D:2 * D, :]) + q3[2 * D:3 * D, :]
        r = r - q
        qsum = qsum + q
        acc += jnp.sum(r * r, axis=(0, 1), keepdims=True)

    out_ref[...] = qsum

    @pl.when(pid == 0)
    def _():
        loss_ref[...] = jnp.zeros((1, 1), jnp.float32)

    loss_ref[...] += acc * (1.25 / (n_tokens * D))


def kernel(x, codebooks):
    n = x.shape[0]
    grid = n // BLOCK
    xt = x.T
    cbt = jnp.swapaxes(codebooks, 1, 2)
    out_t, loss = pl.pallas_call(
        functools.partial(_rvq_body, n_tokens=n),
        grid=(grid,),
        in_specs=[
            pl.BlockSpec((D, BLOCK), lambda i: (0, i)),
            pl.BlockSpec((NUM_Q, K, D), lambda i: (0, 0, 0)),
            pl.BlockSpec((NUM_Q, D, K), lambda i: (0, 0, 0)),
        ],
        out_specs=[
            pl.BlockSpec((D, BLOCK), lambda i: (0, i)),
            pl.BlockSpec((1, 1), lambda i: (0, 0)),
        ],
        out_shape=[
            jax.ShapeDtypeStruct((D, n), jnp.float32),
            jax.ShapeDtypeStruct((1, 1), jnp.float32),
        ],
        compiler_params=pltpu.CompilerParams(
            dimension_semantics=("arbitrary",),
            vmem_limit_bytes=100 * 1024 * 1024,
        ),
    )(xt, codebooks, cbt)
    return out_t.T, loss[0, 0]


# speedup vs baseline: 1.0843x; 1.0843x over previous
import functools

import jax
import jax.numpy as jnp
from jax.experimental import pallas as pl
from jax.experimental.pallas import tpu as pltpu


NUM_Q = 4
K = 1024
D = 32
BLOCK = 2048


def _rvq_body(xt_ref, cb_ref, cbt_ref, out_ref, loss_ref, *, n_tokens):
    pid = pl.program_id(0)
    r = xt_ref[...]
    qsum = jnp.zeros_like(r)
    acc = jnp.zeros((1, 1), jnp.float32)
    iota_k = jax.lax.broadcasted_iota(jnp.int32, (K, 1), 0).astype(jnp.float32)

    for i in range(NUM_Q):
        Wc = cb_ref[i]
        Wt = cbt_ref[i]
        w2 = jnp.sum(Wc * Wc, axis=1, keepdims=True)
        r2 = jnp.sum(r * r, axis=0, keepdims=True)
        r2b = (2.0 * r).astype(jnp.bfloat16)
        scores = jax.lax.dot_general(
            Wc.astype(jnp.bfloat16), r2b, (((1,), (0,)), ((), ())),
            preferred_element_type=jnp.float32)
        dist = (r2 - scores) + w2
        cmin = jnp.min(dist, axis=0, keepdims=True)
        idx = jnp.min(
            jnp.where(dist <= cmin, iota_k, jnp.float32(K)),
            axis=0, keepdims=True)
        onehot = (iota_k == idx).astype(jnp.bfloat16)
        hi = Wt.astype(jnp.bfloat16)
        rem = Wt - hi.astype(jnp.float32)
        mid = rem.astype(jnp.bfloat16)
        lo = (rem - mid.astype(jnp.float32)).astype(jnp.bfloat16)
        wsplit = jnp.concatenate([hi, mid, lo], axis=0)
        q3 = jax.lax.dot_general(
            wsplit, onehot, (((1,), (0,)), ((), ())),
            preferred_element_type=jnp.float32)
        q = (q3[0:D, :] + q3[D:2 * D, :]) + q3[2 * D:3 * D, :]
        r = r - q
        qsum = qsum + q
        acc += jnp.sum(r * r, axis=(0, 1), keepdims=True)

    out_ref[...] = qsum

    @pl.when(pid == 0)
    def _():
        loss_ref[...] = jnp.zeros((1, 1), jnp.float32)

    loss_ref[...] += acc * (1.25 / (n_tokens * D))


def kernel(x, codebooks):
    n = x.shape[0]
    grid = n // BLOCK
    xt = x.T
    cbt = jnp.swapaxes(codebooks, 1, 2)
    out_t, loss = pl.pallas_call(
        functools.partial(_rvq_body, n_tokens=n),
        grid=(grid,),
        in_specs=[
            pl.BlockSpec((D, BLOCK), lambda i: (0, i)),
            pl.BlockSpec((NUM_Q, K, D), lambda i: (0, 0, 0)),
            pl.BlockSpec((NUM_Q, D, K), lambda i: (0, 0, 0)),
        ],
        out_specs=[
            pl.BlockSpec((D, BLOCK), lambda i: (0, i)),
            pl.BlockSpec((1, 1), lambda i: (0, 0)),
        ],
        out_shape=[
            jax.ShapeDtypeStruct((D, n), jnp.float32),
            jax.ShapeDtypeStruct((1, 1), jnp.float32),
        ],
        compiler_params=pltpu.CompilerParams(
            dimension_semantics=("arbitrary",),
            vmem_limit_bytes=100 * 1024 * 1024,
        ),
    )(xt, codebooks, cbt)
    return out_t.T, loss[0, 0]


# speedup vs baseline: 1.1506x; 1.0611x over previous
import functools

import jax
import jax.numpy as jnp
from jax.experimental import pallas as pl
from jax.experimental.pallas import tpu as pltpu


NUM_Q = 4
K = 1024
D = 32
DA = 40
BLOCK = 2048


def _bf16_split3(a):
    hi = a.astype(jnp.bfloat16)
    rem = a - hi.astype(jnp.float32)
    mid = rem.astype(jnp.bfloat16)
    lo = (rem - mid.astype(jnp.float32)).astype(jnp.bfloat16)
    return hi, mid, lo


def _rvq_body(xt_ref, cba_ref, cbt_ref, out_ref, loss_ref, *, n_tokens):
    pid = pl.program_id(0)
    r = xt_ref[...]
    b = r.shape[1]
    qsum = jnp.zeros_like(r)
    acc = jnp.zeros((1, 1), jnp.float32)
    iota_k = jax.lax.broadcasted_iota(jnp.int32, (K, 1), 0).astype(jnp.float32)
    ones3 = jnp.ones((3, b), jnp.bfloat16)
    zeros4 = jnp.zeros((4, b), jnp.bfloat16)

    for i in range(NUM_Q):
        Wt = cbt_ref[i]
        r2 = jnp.sum(r * r, axis=0, keepdims=True)
        rhs = jnp.concatenate(
            [(-2.0 * r).astype(jnp.bfloat16),
             ones3,
             r2.astype(jnp.bfloat16),
             zeros4],
            axis=0)
        dist = jax.lax.dot_general(
            cba_ref[i], rhs, (((1,), (0,)), ((), ())),
            preferred_element_type=jnp.float32)
        cmin = jnp.min(dist, axis=0, keepdims=True)
        idx = jnp.min(
            jnp.where(dist <= cmin, iota_k, jnp.float32(K)),
            axis=0, keepdims=True)
        onehot = (iota_k == idx).astype(jnp.bfloat16)
        hi, mid, lo = _bf16_split3(Wt)
        wsplit = jnp.concatenate([hi, mid, lo], axis=0)
        q3 = jax.lax.dot_general(
            wsplit, onehot, (((1,), (0,)), ((), ())),
            preferred_element_type=jnp.float32)
        q = (q3[0:D, :] + q3[D:2 * D, :]) + q3[2 * D:3 * D, :]
        r = r - q
        qsum = qsum + q
        acc += jnp.sum(r * r, axis=(0, 1), keepdims=True)

    out_ref[...] = qsum

    @pl.when(pid == 0)
    def _():
        loss_ref[...] = jnp.zeros((1, 1), jnp.float32)

    loss_ref[...] += acc * (1.25 / (n_tokens * D))


def kernel(x, codebooks):
    n = x.shape[0]
    grid = n // BLOCK
    xt = x.T
    cbt = jnp.swapaxes(codebooks, 1, 2)
    w2 = jnp.sum(codebooks * codebooks, axis=2, keepdims=True)
    w2h = w2.astype(jnp.bfloat16)
    w2rem = w2 - w2h.astype(jnp.float32)
    w2m = w2rem.astype(jnp.bfloat16)
    w2l = (w2rem - w2m.astype(jnp.float32)).astype(jnp.bfloat16)
    cba = jnp.concatenate(
        [codebooks.astype(jnp.bfloat16), w2h, w2m, w2l,
         jnp.ones((NUM_Q, K, 1), jnp.bfloat16),
         jnp.zeros((NUM_Q, K, 4), jnp.bfloat16)],
        axis=2)
    out_t, loss = pl.pallas_call(
        functools.partial(_rvq_body, n_tokens=n),
        grid=(grid,),
        in_specs=[
            pl.BlockSpec((D, BLOCK), lambda i: (0, i)),
            pl.BlockSpec((NUM_Q, K, DA), lambda i: (0, 0, 0)),
            pl.BlockSpec((NUM_Q, D, K), lambda i: (0, 0, 0)),
        ],
        out_specs=[
            pl.BlockSpec((D, BLOCK), lambda i: (0, i)),
            pl.BlockSpec((1, 1), lambda i: (0, 0)),
        ],
        out_shape=[
            jax.ShapeDtypeStruct((D, n), jnp.float32),
            jax.ShapeDtypeStruct((1, 1), jnp.float32),
        ],
        compiler_params=pltpu.CompilerParams(
            dimension_semantics=("arbitrary",),
            vmem_limit_bytes=100 * 1024 * 1024,
        ),
    )(xt, cba, cbt)
    return out_t.T, loss[0, 0]
